# baseline (speedup 1.0000x reference)
import functools

import jax
import jax.numpy as jnp
from jax import lax
from jax.experimental import pallas as pl
from jax.experimental.pallas import tpu as pltpu
from jax.experimental.pallas import tpu_sc as plsc

NC = 2
NS = 16
NW = NC * NS
L = 16

N = 4096
J = 100
D = 10
R = J * D
IPW = N // NW
KCH = (4, 4, 2)
ILV = IPW // L
BUFR = max(KCH) * J


def _ew_body(x_ref, o_ref):
    o_ref[...] = x_ref[...] * 0.5 + 2.0


@jax.jit
def _elementwise(input0):
    return pl.pallas_call(
        _ew_body,
        out_shape=jax.ShapeDtypeStruct(input0.shape, input0.dtype),
    )(input0)


def _gather_body(idx_hbm, table_hbm, out_hbm, idx_v, tbl_v, out_buf, sems):
    wid = lax.axis_index("s") * NC + lax.axis_index("c")
    i0 = wid * IPW
    pltpu.sync_copy(idx_hbm.at[:, pl.ds(i0, IPW)], idx_v)
    pltpu.sync_copy(table_hbm, tbl_v)

    copies = [None, None]
    k0 = 0
    for c, kch in enumerate(KCH):
        slot = c % 2
        if copies[slot] is not None:
            copies[slot].wait()

        @plsc.parallel_loop(0, J, unroll=4)
        def row(j):
            for il in range(ILV):
                qv = idx_v[j, pl.ds(il * L, L)]
                ta = qv * D
                for kk in range(kch):
                    t = plsc.load_gather(tbl_v, [ta + (k0 + kk)])
                    out_buf[slot, kk * J + j, pl.ds(il * L, L)] = t

        copies[slot] = pltpu.async_copy(
            out_buf.at[slot, pl.ds(0, kch * J)],
            out_hbm.at[pl.ds(k0 * J, kch * J), pl.ds(i0, IPW)],
            sems.at[slot],
        )
        k0 += kch
    for cp in copies:
        if cp is not None:
            cp.wait()


@jax.jit
def _gather(idx_t, table_flat):
    mesh = plsc.VectorSubcoreMesh(core_axis_name="c", subcore_axis_name="s")
    f = functools.partial(
        pl.kernel,
        out_type=jax.ShapeDtypeStruct((R, N), jnp.float32),
        mesh=mesh,
        compiler_params=pltpu.CompilerParams(needs_layout_passes=False),
        scratch_types=[
            pltpu.VMEM((J, IPW), jnp.int32),
            pltpu.VMEM((1024,), jnp.float32),
            pltpu.VMEM((2, BUFR, IPW), jnp.float32),
            pltpu.SemaphoreType.DMA((2,)),
        ],
    )(_gather_body)
    return f(idx_t, table_flat)


def kernel(input0, input1, table):
    output0 = _elementwise(input0)
    idx_t = input1.astype(jnp.int32).T
    table_flat = jnp.pad(table.reshape(-1), (0, 24))
    out_t = _gather(idx_t, table_flat)
    output1 = jnp.transpose(out_t.reshape(D, J, N), (2, 1, 0))
    return (output0, output1)


# speedup vs baseline: 1.0384x; 1.0384x over previous
import functools

import jax
import jax.numpy as jnp
from jax import lax
from jax.experimental import pallas as pl
from jax.experimental.pallas import tpu as pltpu
from jax.experimental.pallas import tpu_sc as plsc

NC = 2
NS = 16
NW = NC * NS
L = 16

N = 4096
J = 100
D = 10
R = J * D
IPW = N // NW
KCH = (4, 4, 2)
ILV = IPW // L
BUFR = max(KCH) * J


def _ew_body(x_ref, o_ref):
    o_ref[...] = x_ref[...] * 0.5 + 2.0


@jax.jit
def _elementwise(input0):
    return pl.pallas_call(
        _ew_body,
        out_shape=jax.ShapeDtypeStruct(input0.shape, input0.dtype),
    )(input0)


def _gather_body(idx_hbm, table_hbm, out_hbm, idx_v, tbl_v, out_buf, sems):
    wid = lax.axis_index("s") * NC + lax.axis_index("c")
    i0 = wid * IPW
    stage_i = pltpu.async_copy(idx_hbm.at[:, pl.ds(i0, IPW)], idx_v, sems.at[0])
    stage_t = pltpu.async_copy(table_hbm, tbl_v, sems.at[1])
    stage_i.wait()
    stage_t.wait()

    copies = [None, None]
    k0 = 0
    for c, kch in enumerate(KCH):
        slot = c % 2
        if copies[slot] is not None:
            copies[slot].wait()

        @plsc.parallel_loop(0, J, unroll=2)
        def row(j):
            for il in range(ILV):
                qv = idx_v[j, pl.ds(il * L, L)]
                ta = qv * D
                for kk in range(kch):
                    t = plsc.load_gather(tbl_v, [ta + (k0 + kk)])
                    out_buf[slot, kk * J + j, pl.ds(il * L, L)] = t

        copies[slot] = pltpu.async_copy(
            out_buf.at[slot, pl.ds(0, kch * J)],
            out_hbm.at[pl.ds(k0 * J, kch * J), pl.ds(i0, IPW)],
            sems.at[slot],
        )
        k0 += kch
    for cp in copies:
        if cp is not None:
            cp.wait()


@jax.jit
def _gather(idx_t, table_flat):
    mesh = plsc.VectorSubcoreMesh(core_axis_name="c", subcore_axis_name="s")
    f = functools.partial(
        pl.kernel,
        out_type=jax.ShapeDtypeStruct((R, N), jnp.float32),
        mesh=mesh,
        compiler_params=pltpu.CompilerParams(needs_layout_passes=False),
        scratch_types=[
            pltpu.VMEM((J, IPW), jnp.int32),
            pltpu.VMEM((1024,), jnp.float32),
            pltpu.VMEM((2, BUFR, IPW), jnp.float32),
            pltpu.SemaphoreType.DMA((2,)),
        ],
    )(_gather_body)
    return f(idx_t, table_flat)


def kernel(input0, input1, table):
    output0 = _elementwise(input0)
    idx_t = input1.astype(jnp.int32).T
    table_flat = jnp.pad(table.reshape(-1), (0, 24))
    out_t = _gather(idx_t, table_flat)
    output1 = jnp.transpose(out_t.reshape(D, J, N), (2, 1, 0))
    return (output0, output1)


# speedup vs baseline: 1.0898x; 1.0495x over previous
import functools

import jax
import jax.numpy as jnp
from jax import lax
from jax.experimental import pallas as pl
from jax.experimental.pallas import tpu as pltpu
from jax.experimental.pallas import tpu_sc as plsc

NC = 2
NS = 16
NW = NC * NS
L = 16

N = 4096
J = 100
D = 10
R = J * D
IPW = N // NW
KCH = (4, 4, 2)
ILV = IPW // L
BUFR = max(KCH) * J


def _ew_body(x_ref, o_ref):
    o_ref[...] = x_ref[...] * 0.5 + 2.0


@jax.jit
def _elementwise(input0):
    return pl.pallas_call(
        _ew_body,
        out_shape=jax.ShapeDtypeStruct(input0.shape, input0.dtype),
    )(input0)


def _gather_body(idx_hbm, table_hbm, out_hbm, idx_v, tbl_v, out_buf, sems):
    wid = lax.axis_index("s") * NC + lax.axis_index("c")
    i0 = wid * IPW
    stage_i = pltpu.async_copy(idx_hbm.at[:, pl.ds(i0, IPW)], idx_v, sems.at[0])
    stage_t = pltpu.async_copy(table_hbm, tbl_v, sems.at[1])
    stage_i.wait()
    stage_t.wait()

    copies = [None, None]
    k0 = 0
    for c, kch in enumerate(KCH):
        slot = c % 2
        if copies[slot] is not None:
            copies[slot].wait()

        @plsc.parallel_loop(0, J, unroll=1)
        def row(j):
            for il in range(ILV):
                qv = idx_v[j, pl.ds(il * L, L)]
                ta = qv * D
                for kk in range(kch):
                    t = plsc.load_gather(tbl_v, [ta + (k0 + kk)])
                    out_buf[slot, kk * J + j, pl.ds(il * L, L)] = t

        copies[slot] = pltpu.async_copy(
            out_buf.at[slot, pl.ds(0, kch * J)],
            out_hbm.at[pl.ds(k0 * J, kch * J), pl.ds(i0, IPW)],
            sems.at[slot],
        )
        k0 += kch
    for cp in copies:
        if cp is not None:
            cp.wait()


@jax.jit
def _gather(idx_t, table_flat):
    mesh = plsc.VectorSubcoreMesh(core_axis_name="c", subcore_axis_name="s")
    f = functools.partial(
        pl.kernel,
        out_type=jax.ShapeDtypeStruct((R, N), jnp.float32),
        mesh=mesh,
        compiler_params=pltpu.CompilerParams(needs_layout_passes=False),
        scratch_types=[
            pltpu.VMEM((J, IPW), jnp.int32),
            pltpu.VMEM((1024,), jnp.float32),
            pltpu.VMEM((2, BUFR, IPW), jnp.float32),
            pltpu.SemaphoreType.DMA((2,)),
        ],
    )(_gather_body)
    return f(idx_t, table_flat)


def kernel(input0, input1, table):
    output0 = _elementwise(input0)
    idx_t = input1.astype(jnp.int32).T
    table_flat = jnp.pad(table.reshape(-1), (0, 24))
    out_t = _gather(idx_t, table_flat)
    output1 = jnp.transpose(out_t.reshape(D, J, N), (2, 1, 0))
    return (output0, output1)
